# Initial kernel scaffold; baseline (speedup 1.0000x reference)
import jax
import jax.numpy as jnp
from jax.experimental import pallas as pl


def kernel(x, pe_table, position_ids):
    raise NotImplementedError("write your pallas kernel here")


# speedup vs baseline: 1.5428x; 1.5428x over previous
import functools

import jax
import jax.numpy as jnp
from jax import lax
from jax.experimental import pallas as pl
from jax.experimental.pallas import tpu as pltpu
from jax.experimental.pallas import tpu_sc as plsc

_SEQ = 8192
_D = 1024
_NC, _NS = 2, 16
_NW = _NC * _NS
_BPW = _SEQ // _NW
_CH = 32
_NCHUNK = _BPW // _CH

_mesh = plsc.VectorSubcoreMesh(core_axis_name="c", subcore_axis_name="s")


@functools.partial(
    pl.kernel,
    out_type=jax.ShapeDtypeStruct((_SEQ, _D), jnp.float32),
    mesh=_mesh,
    scratch_types=[
        pltpu.VMEM((_BPW,), jnp.int32),
        pltpu.VMEM((_CH, _D), jnp.float32),
        pltpu.VMEM((_CH, _D), jnp.float32),
        pltpu.SemaphoreType.DMA,
        pltpu.SemaphoreType.DMA,
    ],
)
def _pe_gather(table_hbm, idx_hbm, out_hbm, idx_v, buf0, buf1, sem0, sem1):
    wid = lax.axis_index("s") * _NC + lax.axis_index("c")
    base = wid * _BPW
    pltpu.sync_copy(idx_hbm.at[pl.ds(base, _BPW)], idx_v)

    bufs = (buf0, buf1)
    sems = (sem0, sem1)
    copies = [None, None]
    copies[0] = pltpu.async_copy(
        table_hbm.at[idx_v.at[pl.ds(0, _CH)]], bufs[0], sems[0])
    for c in range(_NCHUNK):
        cur = c % 2
        nxt = (c + 1) % 2
        if c + 1 < _NCHUNK:
            copies[nxt] = pltpu.async_copy(
                table_hbm.at[idx_v.at[pl.ds((c + 1) * _CH, _CH)]],
                bufs[nxt], sems[nxt])
        copies[cur].wait()
        pltpu.sync_copy(bufs[cur], out_hbm.at[pl.ds(base + c * _CH, _CH)])


def kernel(x, pe_table, position_ids):
    del x
    idx = position_ids.reshape(_SEQ).astype(jnp.int32)
    out = _pe_gather(pe_table, idx)
    return out.reshape(1, _SEQ, _D)


# speedup vs baseline: 1.5881x; 1.0294x over previous
import functools

import jax
import jax.numpy as jnp
from jax import lax
from jax.experimental import pallas as pl
from jax.experimental.pallas import tpu as pltpu
from jax.experimental.pallas import tpu_sc as plsc

_SEQ = 8192
_D = 1024
_NC, _NS = 2, 16
_NW = _NC * _NS
_BPW = _SEQ // _NW
_CH = 32
_NCHUNK = _BPW // _CH
_NBUF = 3

_mesh = plsc.VectorSubcoreMesh(core_axis_name="c", subcore_axis_name="s")


@functools.partial(
    pl.kernel,
    out_type=jax.ShapeDtypeStruct((_SEQ, _D), jnp.float32),
    mesh=_mesh,
    scratch_types=[
        pltpu.VMEM((_BPW,), jnp.int32),
        [pltpu.VMEM((_CH, _D), jnp.float32) for _ in range(_NBUF)],
        [pltpu.SemaphoreType.DMA for _ in range(_NBUF)],
        [pltpu.SemaphoreType.DMA for _ in range(_NBUF)],
    ],
)
def _pe_gather(table_hbm, idx_hbm, out_hbm, idx_v, bufs, gsems, ssems):
    wid = lax.axis_index("s") * _NC + lax.axis_index("c")
    base = wid * _BPW
    pltpu.sync_copy(idx_hbm.at[pl.ds(base, _BPW)], idx_v)

    gathers = [None] * _NBUF
    stores = [None] * _NBUF
    for c in range(min(_NBUF, _NCHUNK)):
        gathers[c] = pltpu.async_copy(
            table_hbm.at[idx_v.at[pl.ds(c * _CH, _CH)]], bufs[c], gsems[c])
    for c in range(_NCHUNK):
        b = c % _NBUF
        gathers[b].wait()
        stores[b] = pltpu.async_copy(
            bufs[b], out_hbm.at[pl.ds(base + c * _CH, _CH)], ssems[b])
        nc = c + _NBUF
        if nc < _NCHUNK:
            stores[b].wait()
            gathers[b] = pltpu.async_copy(
                table_hbm.at[idx_v.at[pl.ds(nc * _CH, _CH)]], bufs[b], gsems[b])
    for c in range(max(0, _NCHUNK - _NBUF), _NCHUNK):
        b = c % _NBUF
        stores[b].wait()


def kernel(x, pe_table, position_ids):
    del x
    idx = position_ids.reshape(_SEQ).astype(jnp.int32)
    out = _pe_gather(pe_table, idx)
    return out.reshape(1, _SEQ, _D)


# speedup vs baseline: 1.5947x; 1.0042x over previous
import functools

import jax
import jax.numpy as jnp
from jax import lax
from jax.experimental import pallas as pl
from jax.experimental.pallas import tpu as pltpu
from jax.experimental.pallas import tpu_sc as plsc

_SEQ = 8192
_D = 1024
_NC, _NS = 2, 16
_NW = _NC * _NS
_BPW = _SEQ // _NW
_CH = 16
_NCHUNK = _BPW // _CH
_NBUF = 6

_mesh = plsc.VectorSubcoreMesh(core_axis_name="c", subcore_axis_name="s")


@functools.partial(
    pl.kernel,
    out_type=jax.ShapeDtypeStruct((_SEQ, _D), jnp.float32),
    mesh=_mesh,
    scratch_types=[
        pltpu.VMEM((_BPW,), jnp.int32),
        [pltpu.VMEM((_CH, _D), jnp.float32) for _ in range(_NBUF)],
        [pltpu.SemaphoreType.DMA for _ in range(_NBUF)],
        [pltpu.SemaphoreType.DMA for _ in range(_NBUF)],
    ],
)
def _pe_gather(table_hbm, idx_hbm, out_hbm, idx_v, bufs, gsems, ssems):
    wid = lax.axis_index("s") * _NC + lax.axis_index("c")
    base = wid * _BPW
    pltpu.sync_copy(idx_hbm.at[pl.ds(base, _BPW)], idx_v)

    gathers = [None] * _NBUF
    stores = [None] * _NBUF
    for c in range(min(_NBUF, _NCHUNK)):
        gathers[c] = pltpu.async_copy(
            table_hbm.at[idx_v.at[pl.ds(c * _CH, _CH)]], bufs[c], gsems[c])
    for c in range(_NCHUNK):
        b = c % _NBUF
        gathers[b].wait()
        stores[b] = pltpu.async_copy(
            bufs[b], out_hbm.at[pl.ds(base + c * _CH, _CH)], ssems[b])
        nc = c + _NBUF
        if nc < _NCHUNK:
            stores[b].wait()
            gathers[b] = pltpu.async_copy(
                table_hbm.at[idx_v.at[pl.ds(nc * _CH, _CH)]], bufs[b], gsems[b])
    for c in range(max(0, _NCHUNK - _NBUF), _NCHUNK):
        b = c % _NBUF
        stores[b].wait()


def kernel(x, pe_table, position_ids):
    del x
    idx = position_ids.reshape(_SEQ).astype(jnp.int32)
    out = _pe_gather(pe_table, idx)
    return out.reshape(1, _SEQ, _D)
